# baseline (speedup 1.0000x reference)
import functools
import math

import jax
import jax.numpy as jnp
from jax import lax
from jax.experimental import pallas as pl
from jax.experimental.pallas import tpu as pltpu
from jax.experimental.pallas import tpu_sc as plsc

_NUM_BUCKETS = 64
_MAX_DISTANCE = 32768
_TABLE_ROWS = 256
_TABLE_COLS = 128

_L = 16
_NW = 32
_N = 64 * 32768
_PER_W = _N // _NW
_CH = 16384
_NCH = _PER_W // _CH


def _table_body(bias_ref, out_ref):
    r = lax.broadcasted_iota(jnp.int32, (_TABLE_ROWS, _TABLE_COLS), 0)
    c = lax.broadcasted_iota(jnp.int32, (_TABLE_ROWS, _TABLE_COLS), 1)
    p = r * _TABLE_COLS + c
    exact = _NUM_BUCKETS // 2
    rel = p.astype(jnp.float32) - exact
    log_b = exact + (_NUM_BUCKETS - exact - 1) * jnp.log(
        jnp.clip(rel, 1.0, None)) / math.log(max(_MAX_DISTANCE - exact, 2))
    bucket = jnp.where(p < exact, p, log_b.astype(jnp.int32))
    bucket = jnp.clip(bucket, 0, _NUM_BUCKETS - 1)
    acc = jnp.zeros((_TABLE_ROWS, _TABLE_COLS), jnp.float32)
    for b in range(_NUM_BUCKETS):
        acc = jnp.where(bucket == b, bias_ref[b], acc)
    out_ref[...] = acc


def _build_table(bias):
    table2d = pl.pallas_call(
        _table_body,
        out_shape=jax.ShapeDtypeStruct((_TABLE_ROWS, _TABLE_COLS), jnp.float32),
        in_specs=[pl.BlockSpec(memory_space=pltpu.SMEM)],
        out_specs=pl.BlockSpec(memory_space=pltpu.VMEM),
    )(bias)
    return table2d.reshape(_MAX_DISTANCE)


_ROWS = 64
_COLS = 32768
_ROWS_PER_W = _ROWS // _NW
_CH_PER_ROW = _COLS // _CH
_NCHUNK = _ROWS_PER_W * _CH_PER_ROW


def _sc_gather(table, positions):
    mesh = plsc.VectorSubcoreMesh(core_axis_name="c", subcore_axis_name="s")

    @functools.partial(
        pl.kernel,
        mesh=mesh,
        out_type=jax.ShapeDtypeStruct((_ROWS, _COLS), jnp.float32),
        compiler_params=pltpu.CompilerParams(
            needs_layout_passes=False,
            disable_bounds_checks=True,
            disable_semaphore_checks=True,
        ),
        scratch_types=[
            pltpu.VMEM((_MAX_DISTANCE,), jnp.float32),
            pltpu.VMEM((_CH,), jnp.int32),
            pltpu.VMEM((_CH,), jnp.int32),
            pltpu.VMEM((_CH,), jnp.float32),
            pltpu.VMEM((_CH,), jnp.float32),
            pltpu.SemaphoreType.DMA,
            pltpu.SemaphoreType.DMA,
            pltpu.SemaphoreType.DMA,
            pltpu.SemaphoreType.DMA,
            pltpu.SemaphoreType.DMA,
        ],
    )
    def k(table_hbm, pos_hbm, out_hbm, table_v, idx0, idx1, val0, val1,
          tsem, lsem0, lsem1, ssem0, ssem1):
        wid = lax.axis_index("s") * 2 + lax.axis_index("c")
        idx = [idx0, idx1]
        val = [val0, val1]
        lsem = [lsem0, lsem1]
        ssem = [ssem0, ssem1]

        def pos_slice(c):
            row = wid * _ROWS_PER_W + (c // _CH_PER_ROW)
            col = (c % _CH_PER_ROW) * _CH
            return (row, pl.ds(col, _CH))

        ht = pltpu.async_copy(table_hbm, table_v, tsem)
        hl = [None, None]
        hs = [None, None]
        for c in range(min(2, _NCHUNK)):
            r, cs = pos_slice(c)
            hl[c % 2] = pltpu.async_copy(pos_hbm.at[r, cs], idx[c % 2], lsem[c % 2])

        for c in range(_NCHUNK):
            b = c % 2
            hl[b].wait()
            if c == 0:
                ht.wait()
            if hs[b] is not None:
                hs[b].wait()

            @plsc.parallel_loop(0, _CH, step=_L, unroll=32)
            def gather_body(i, _idx=idx[b], _val=val[b]):
                _val[pl.ds(i, _L)] = plsc.load_gather(table_v, [_idx[pl.ds(i, _L)]])

            r, cs = pos_slice(c)
            hs[b] = pltpu.async_copy(val[b], out_hbm.at[r, cs], ssem[b])
            if c + 2 < _NCHUNK:
                r2, cs2 = pos_slice(c + 2)
                hl[b] = pltpu.async_copy(pos_hbm.at[r2, cs2], idx[b], lsem[b])

        for h in hs:
            if h is not None:
                h.wait()

    return k(table, positions)


def kernel(positions, bias):
    table = _build_table(bias)
    return _sc_gather(table, positions)


# speedup vs baseline: 1.0156x; 1.0156x over previous
import functools
import math

import jax
import jax.numpy as jnp
from jax import lax
from jax.experimental import pallas as pl
from jax.experimental.pallas import tpu as pltpu
from jax.experimental.pallas import tpu_sc as plsc

_NUM_BUCKETS = 64
_MAX_DISTANCE = 32768
_TABLE_ROWS = 256
_TABLE_COLS = 128

_L = 16
_NW = 32
_N = 64 * 32768
_PER_W = _N // _NW
_CH = 16384
_NCH = _PER_W // _CH


def _table_body(bias_ref, out_ref):
    r = lax.broadcasted_iota(jnp.int32, (_TABLE_ROWS, _TABLE_COLS), 0)
    c = lax.broadcasted_iota(jnp.int32, (_TABLE_ROWS, _TABLE_COLS), 1)
    p = r * _TABLE_COLS + c
    exact = _NUM_BUCKETS // 2
    rel = p.astype(jnp.float32) - exact
    log_b = exact + (_NUM_BUCKETS - exact - 1) * jnp.log(
        jnp.clip(rel, 1.0, None)) / math.log(max(_MAX_DISTANCE - exact, 2))
    bucket = jnp.where(p < exact, p, log_b.astype(jnp.int32))
    bucket = jnp.clip(bucket, 0, _NUM_BUCKETS - 1)
    acc = jnp.zeros((_TABLE_ROWS, _TABLE_COLS), jnp.float32)
    for b in range(_NUM_BUCKETS):
        acc = jnp.where(bucket == b, bias_ref[b], acc)
    out_ref[...] = acc


def _build_table(bias):
    table2d = pl.pallas_call(
        _table_body,
        out_shape=jax.ShapeDtypeStruct((_TABLE_ROWS, _TABLE_COLS), jnp.float32),
        in_specs=[pl.BlockSpec(memory_space=pltpu.SMEM)],
        out_specs=pl.BlockSpec(memory_space=pltpu.VMEM),
    )(bias)
    return table2d.reshape(_MAX_DISTANCE)


_ROWS = 64
_COLS = 32768
_ROWS_PER_W = _ROWS // _NW
_CH_PER_ROW = _COLS // _CH
_NCHUNK = _ROWS_PER_W * _CH_PER_ROW


def _sc_gather(table, positions):
    mesh = plsc.VectorSubcoreMesh(core_axis_name="c", subcore_axis_name="s")

    @functools.partial(
        pl.kernel,
        mesh=mesh,
        out_type=jax.ShapeDtypeStruct((_ROWS, _COLS), jnp.float32),
        compiler_params=pltpu.CompilerParams(
            needs_layout_passes=False,
            disable_bounds_checks=True,
            disable_semaphore_checks=True,
        ),
        scratch_types=[
            pltpu.VMEM((_MAX_DISTANCE,), jnp.float32),
            pltpu.VMEM((_CH,), jnp.int32),
            pltpu.VMEM((_CH,), jnp.int32),
            pltpu.VMEM((_CH,), jnp.float32),
            pltpu.VMEM((_CH,), jnp.float32),
            pltpu.SemaphoreType.DMA,
            pltpu.SemaphoreType.DMA,
            pltpu.SemaphoreType.DMA,
            pltpu.SemaphoreType.DMA,
            pltpu.SemaphoreType.DMA,
        ],
    )
    def k(table_hbm, pos_hbm, out_hbm, table_v, idx0, idx1, val0, val1,
          tsem, lsem0, lsem1, ssem0, ssem1):
        wid = lax.axis_index("s") * 2 + lax.axis_index("c")
        idx = [idx0, idx1]
        val = [val0, val1]
        lsem = [lsem0, lsem1]
        ssem = [ssem0, ssem1]

        def pos_slice(c):
            row = wid * _ROWS_PER_W + (c // _CH_PER_ROW)
            col = (c % _CH_PER_ROW) * _CH
            return (row, pl.ds(col, _CH))

        ht = pltpu.async_copy(table_hbm, table_v, tsem)
        hl = [None, None]
        hs = [None, None]
        for c in range(min(2, _NCHUNK)):
            r, cs = pos_slice(c)
            hl[c % 2] = pltpu.async_copy(pos_hbm.at[r, cs], idx[c % 2], lsem[c % 2])

        for c in range(_NCHUNK):
            b = c % 2
            hl[b].wait()
            if c == 0:
                ht.wait()
            if hs[b] is not None:
                hs[b].wait()

            @plsc.parallel_loop(0, _CH, step=_L, unroll=16)
            def gather_body(i, _idx=idx[b], _val=val[b]):
                _val[pl.ds(i, _L)] = plsc.load_gather(table_v, [_idx[pl.ds(i, _L)]])

            r, cs = pos_slice(c)
            hs[b] = pltpu.async_copy(val[b], out_hbm.at[r, cs], ssem[b])
            if c + 2 < _NCHUNK:
                r2, cs2 = pos_slice(c + 2)
                hl[b] = pltpu.async_copy(pos_hbm.at[r2, cs2], idx[b], lsem[b])

        for h in hs:
            if h is not None:
                h.wait()

    return k(table, positions)


def kernel(positions, bias):
    table = _build_table(bias)
    return _sc_gather(table, positions)
